# Initial kernel scaffold; baseline (speedup 1.0000x reference)
import jax
import jax.numpy as jnp
from jax.experimental import pallas as pl


def kernel(X, B, edge_index, rev_index, batch_vec, num_graphs, W_i, W_h, W_a):
    raise NotImplementedError("write your pallas kernel here")


# speedup vs baseline: 1.6005x; 1.6005x over previous
import functools

import jax
import jax.numpy as jnp
from jax import lax
from jax.experimental import pallas as pl
from jax.experimental.pallas import tpu as pltpu
from jax.experimental.pallas import tpu_sc as plsc

N_NODES = 10000
N_EDGES = 320000
D = 128
D_EDGE = 16
STEPS = 3
NUM_GRAPHS = 64

NC = 2
NS = 16
NW = NC * NS
L = 16

CH = 128
NPAD = 10240
NTG = 128


def _sc_mesh():
    return plsc.VectorSubcoreMesh(
        core_axis_name="c", subcore_axis_name="s", num_cores=NC, num_subcores=NS
    )


def _mm(a, wt, block_m=512):
    M, K = a.shape
    N = wt.shape[1]

    def body(a_ref, w_ref, o_ref):
        o_ref[...] = jnp.dot(a_ref[...], w_ref[...],
                             preferred_element_type=jnp.float32)

    return pl.pallas_call(
        body,
        grid=(M // block_m,),
        in_specs=[
            pl.BlockSpec((block_m, K), lambda i: (i, 0)),
            pl.BlockSpec((K, N), lambda i: (0, 0)),
        ],
        out_specs=pl.BlockSpec((block_m, N), lambda i: (i, 0)),
        out_shape=jax.ShapeDtypeStruct((M, N), jnp.float32),
    )(a, wt)


def _mm2_relu(x, wx, parts, wh, block_m=512):
    M, K = x.shape
    N = wx.shape[1]

    def body(x_ref, wx_ref, p_ref, wh_ref, o_ref):
        h = p_ref[0] + p_ref[1]
        acc = jnp.dot(x_ref[...], wx_ref[...],
                      preferred_element_type=jnp.float32)
        acc = acc + jnp.dot(h, wh_ref[...],
                            preferred_element_type=jnp.float32)
        o_ref[...] = jnp.maximum(acc, 0.0)

    return pl.pallas_call(
        body,
        grid=(M // block_m,),
        in_specs=[
            pl.BlockSpec((block_m, K), lambda i: (i, 0)),
            pl.BlockSpec((K, N), lambda i: (0, 0)),
            pl.BlockSpec((2, block_m, D), lambda i: (0, i, 0)),
            pl.BlockSpec((D, N), lambda i: (0, 0)),
        ],
        out_specs=pl.BlockSpec((block_m, N), lambda i: (i, 0)),
        out_shape=jax.ShapeDtypeStruct((M, N), jnp.float32),
    )(x, wx, parts, wh)


def _add2(parts, block_m):
    _, NT, D_ = parts.shape

    def body(p_ref, o_ref):
        o_ref[...] = p_ref[0] + p_ref[1]

    return pl.pallas_call(
        body,
        grid=(NT // block_m,),
        in_specs=[pl.BlockSpec((2, block_m, D_), lambda i: (0, i, 0))],
        out_specs=pl.BlockSpec((block_m, D_), lambda i: (i, 0)),
        out_shape=jax.ShapeDtypeStruct((NT, D_), jnp.float32),
    )(parts)


def _scatter_add_call(rows, idx, zeros, nt):
    e_rows = rows.shape[0]
    n_chunks = e_rows // CH
    jmax = -(-n_chunks // NW)
    rpt = nt // NS

    def body(rows_hbm, idx_hbm, zeros_hbm, out_hbm, idx_v, rows_v, table_sh, sem):
        c = lax.axis_index("c")
        s = lax.axis_index("s")
        wid = s * NC + c
        pltpu.sync_copy(zeros_hbm, table_sh.at[pl.ds(s * rpt, rpt)])
        plsc.subcore_barrier()

        def step(j, carry):
            cid = wid + NW * j

            @pl.when(cid < n_chunks)
            def _():
                off = cid * CH
                pltpu.sync_copy(idx_hbm.at[pl.ds(off, CH)], idx_v)
                pltpu.async_copy(rows_hbm.at[pl.ds(off, CH)], rows_v, sem).wait()
                pltpu.sync_copy(rows_v, table_sh.at[idx_v], add=True)

            return carry

        lax.fori_loop(0, jmax, step, 0)
        plsc.subcore_barrier()
        pltpu.sync_copy(
            table_sh.at[pl.ds(s * rpt, rpt)],
            out_hbm.at[pl.ds(c * nt + s * rpt, rpt)],
        )

    f = pl.kernel(
        body,
        out_type=jax.ShapeDtypeStruct((NC * nt, D), jnp.float32),
        mesh=_sc_mesh(),
        scratch_types=[
            pltpu.VMEM((CH,), jnp.int32),
            pltpu.VMEM((CH, D), jnp.float32),
            pltpu.VMEM_SHARED((nt, D), jnp.float32),
            pltpu.SemaphoreType.DMA,
        ],
    )
    return f(rows, idx, zeros)


def _fused_gather_call(lin, t1, i1, t2=None, i2=None):
    e_rows = lin.shape[0]
    n_chunks = e_rows // CH
    jmax = -(-n_chunks // NW)
    has_sub = t2 is not None

    def compute(c_v, a_v, b_v):
        def rbody(r, carry):
            for q in range(D // L):
                sl = pl.ds(q * L, L)
                v = c_v[r, sl] + a_v[r, sl]
                if b_v is not None:
                    v = v - b_v[r, sl]
                c_v[r, sl] = jnp.maximum(v, 0.0)
            return carry

        lax.fori_loop(0, CH, rbody, 0)

    if has_sub:
        def body(lin_hbm, t1_hbm, i1_hbm, t2_hbm, i2_hbm, out_hbm,
                 i1_v, i2_v, a_v, b_v, c_v, sem1, sem2, sem3):
            wid = lax.axis_index("s") * NC + lax.axis_index("c")

            def step(j, carry):
                cid = wid + NW * j

                @pl.when(cid < n_chunks)
                def _():
                    off = cid * CH
                    pltpu.sync_copy(i1_hbm.at[pl.ds(off, CH)], i1_v)
                    pltpu.sync_copy(i2_hbm.at[pl.ds(off, CH)], i2_v)
                    cp1 = pltpu.async_copy(t1_hbm.at[i1_v], a_v, sem1)
                    cp2 = pltpu.async_copy(t2_hbm.at[i2_v], b_v, sem2)
                    cp3 = pltpu.async_copy(lin_hbm.at[pl.ds(off, CH)], c_v, sem3)
                    cp3.wait()
                    cp1.wait()
                    cp2.wait()
                    compute(c_v, a_v, b_v)
                    pltpu.sync_copy(c_v, out_hbm.at[pl.ds(off, CH)])

                return carry

            lax.fori_loop(0, jmax, step, 0)

        scratch = [
            pltpu.VMEM((CH,), jnp.int32),
            pltpu.VMEM((CH,), jnp.int32),
            pltpu.VMEM((CH, D), jnp.float32),
            pltpu.VMEM((CH, D), jnp.float32),
            pltpu.VMEM((CH, D), jnp.float32),
            pltpu.SemaphoreType.DMA,
            pltpu.SemaphoreType.DMA,
            pltpu.SemaphoreType.DMA,
        ]
        args = (lin, t1, i1, t2, i2)
    else:
        def body(lin_hbm, t1_hbm, i1_hbm, out_hbm, i1_v, a_v, c_v, sem1, sem3):
            wid = lax.axis_index("s") * NC + lax.axis_index("c")

            def step(j, carry):
                cid = wid + NW * j

                @pl.when(cid < n_chunks)
                def _():
                    off = cid * CH
                    pltpu.sync_copy(i1_hbm.at[pl.ds(off, CH)], i1_v)
                    cp1 = pltpu.async_copy(t1_hbm.at[i1_v], a_v, sem1)
                    cp3 = pltpu.async_copy(lin_hbm.at[pl.ds(off, CH)], c_v, sem3)
                    cp3.wait()
                    cp1.wait()
                    compute(c_v, a_v, None)
                    pltpu.sync_copy(c_v, out_hbm.at[pl.ds(off, CH)])

                return carry

            lax.fori_loop(0, jmax, step, 0)

        scratch = [
            pltpu.VMEM((CH,), jnp.int32),
            pltpu.VMEM((CH, D), jnp.float32),
            pltpu.VMEM((CH, D), jnp.float32),
            pltpu.SemaphoreType.DMA,
            pltpu.SemaphoreType.DMA,
        ]
        args = (lin, t1, i1)

    f = pl.kernel(
        body,
        out_type=jax.ShapeDtypeStruct((e_rows, D), jnp.float32),
        mesh=_sc_mesh(),
        scratch_types=scratch,
    )
    return f(*args)


def kernel(X, B, edge_index, rev_index, batch_vec, num_graphs, W_i, W_h, W_a):
    f32 = jnp.float32
    src = edge_index[0]
    rcv = edge_index[1]

    X_pad = jnp.zeros((NPAD, D), f32).at[:N_NODES].set(X)
    W_ixT = W_i[:, :D].T
    W_ibT = W_i[:, D:].T
    W_hT = W_h.T
    W_axT = W_a[:, :D].T
    W_ahT = W_a[:, D:].T
    z_node = jnp.zeros((NPAD // NS, D), f32)
    z_graph = jnp.zeros((NTG // NS, D), f32)

    XW = _mm(X_pad, W_ixT)
    BW = _mm(B, W_ibT)
    H0 = _fused_gather_call(BW, XW, src)
    H = H0
    for _ in range(STEPS):
        G = _mm(H, W_hT)
        parts = _scatter_add_call(G, rcv, z_node, NPAD).reshape(NC, NPAD, D)
        inc = _add2(parts, 512)
        H = _fused_gather_call(H0, inc, src, G, rev_index)

    parts_f = _scatter_add_call(H, rcv, z_node, NPAD).reshape(NC, NPAD, D)
    P = _mm2_relu(X_pad, W_axT, parts_f, W_ahT)
    b_pad = jnp.concatenate(
        [batch_vec, jnp.zeros((NPAD - N_NODES,), jnp.int32)]
    )
    parts_z = _scatter_add_call(P, b_pad, z_graph, NTG).reshape(NC, NTG, D)
    Z = _add2(parts_z, NTG)[:NUM_GRAPHS]
    return Z + jnp.asarray(num_graphs - NUM_GRAPHS, f32)


# speedup vs baseline: 2.1205x; 1.3249x over previous
import functools

import jax
import jax.numpy as jnp
from jax import lax
from jax.experimental import pallas as pl
from jax.experimental.pallas import tpu as pltpu
from jax.experimental.pallas import tpu_sc as plsc

N_NODES = 10000
N_EDGES = 320000
D = 128
D_EDGE = 16
STEPS = 3
NUM_GRAPHS = 64

NC = 2
NS = 16
NW = NC * NS
L = 16

CH = 128
NPAD = 10240
NTG = 128


def _sc_mesh():
    return plsc.VectorSubcoreMesh(
        core_axis_name="c", subcore_axis_name="s", num_cores=NC, num_subcores=NS
    )


def _mm(a, wt, block_m=512):
    M, K = a.shape
    N = wt.shape[1]

    def body(a_ref, w_ref, o_ref):
        o_ref[...] = jnp.dot(a_ref[...], w_ref[...],
                             preferred_element_type=jnp.float32)

    return pl.pallas_call(
        body,
        grid=(M // block_m,),
        in_specs=[
            pl.BlockSpec((block_m, K), lambda i: (i, 0)),
            pl.BlockSpec((K, N), lambda i: (0, 0)),
        ],
        out_specs=pl.BlockSpec((block_m, N), lambda i: (i, 0)),
        out_shape=jax.ShapeDtypeStruct((M, N), jnp.float32),
    )(a, wt)


def _mm2_relu(x, wx, parts, wh, block_m=512):
    M, K = x.shape
    N = wx.shape[1]

    def body(x_ref, wx_ref, p_ref, wh_ref, o_ref):
        h = p_ref[0] + p_ref[1]
        acc = jnp.dot(x_ref[...], wx_ref[...],
                      preferred_element_type=jnp.float32)
        acc = acc + jnp.dot(h, wh_ref[...],
                            preferred_element_type=jnp.float32)
        o_ref[...] = jnp.maximum(acc, 0.0)

    return pl.pallas_call(
        body,
        grid=(M // block_m,),
        in_specs=[
            pl.BlockSpec((block_m, K), lambda i: (i, 0)),
            pl.BlockSpec((K, N), lambda i: (0, 0)),
            pl.BlockSpec((2, block_m, D), lambda i: (0, i, 0)),
            pl.BlockSpec((D, N), lambda i: (0, 0)),
        ],
        out_specs=pl.BlockSpec((block_m, N), lambda i: (i, 0)),
        out_shape=jax.ShapeDtypeStruct((M, N), jnp.float32),
    )(x, wx, parts, wh)


def _add2(parts, block_m):
    _, NT, D_ = parts.shape

    def body(p_ref, o_ref):
        o_ref[...] = p_ref[0] + p_ref[1]

    return pl.pallas_call(
        body,
        grid=(NT // block_m,),
        in_specs=[pl.BlockSpec((2, block_m, D_), lambda i: (0, i, 0))],
        out_specs=pl.BlockSpec((block_m, D_), lambda i: (i, 0)),
        out_shape=jax.ShapeDtypeStruct((NT, D_), jnp.float32),
    )(parts)


def _scatter_add_call(rows, idx, zeros, nt):
    e_rows = rows.shape[0]
    n_chunks = e_rows // CH
    jmax = -(-n_chunks // NW)
    rpt = nt // NS

    def body(rows_hbm, idx_hbm, zeros_hbm, out_hbm, idx_v, rows_v, table_sh,
             sem0, sem1):
        c = lax.axis_index("c")
        s = lax.axis_index("s")
        wid = s * NC + c
        pltpu.sync_copy(zeros_hbm, table_sh.at[pl.ds(s * rpt, rpt)])
        plsc.subcore_barrier()
        sems = (sem0, sem1)

        def issue(j, p):
            cid = wid + NW * j

            @pl.when(cid < n_chunks)
            def _():
                off = cid * CH
                pltpu.sync_copy(idx_hbm.at[pl.ds(off, CH)], idx_v.at[p])
                pltpu.async_copy(rows_hbm.at[pl.ds(off, CH)], rows_v.at[p],
                                 sems[p])

        def process(j, q):
            cid = wid + NW * j

            @pl.when(cid < n_chunks)
            def _():
                off = cid * CH
                pltpu.make_async_copy(rows_hbm.at[pl.ds(off, CH)],
                                      rows_v.at[q], sems[q]).wait()
                pltpu.sync_copy(rows_v.at[q], table_sh.at[idx_v.at[q]],
                                add=True)

        def step(jj, carry):
            j = 2 * jj
            issue(j + 1, 1)
            process(j, 0)
            issue(j + 2, 0)
            process(j + 1, 1)
            return carry

        issue(0, 0)
        lax.fori_loop(0, (jmax + 1) // 2 + 1, step, 0)
        plsc.subcore_barrier()
        pltpu.sync_copy(
            table_sh.at[pl.ds(s * rpt, rpt)],
            out_hbm.at[pl.ds(c * nt + s * rpt, rpt)],
        )

    f = pl.kernel(
        body,
        out_type=jax.ShapeDtypeStruct((NC * nt, D), jnp.float32),
        mesh=_sc_mesh(),
        scratch_types=[
            pltpu.VMEM((2, CH), jnp.int32),
            pltpu.VMEM((2, CH, D), jnp.float32),
            pltpu.VMEM_SHARED((nt, D), jnp.float32),
            pltpu.SemaphoreType.DMA,
            pltpu.SemaphoreType.DMA,
        ],
    )
    return f(rows, idx, zeros)


def _fused_gather_call(lin, t1, i1, t2=None, i2=None):
    e_rows = lin.shape[0]
    n_chunks = e_rows // CH
    jmax = -(-n_chunks // NW)
    has_sub = t2 is not None

    def compute(c_v, a_v, b_v):
        def rbody(r, carry):
            for q in range(D // L):
                sl = pl.ds(q * L, L)
                v = c_v[r, sl] + a_v[r, sl]
                if b_v is not None:
                    v = v - b_v[r, sl]
                c_v[r, sl] = jnp.maximum(v, 0.0)
            return carry

        lax.fori_loop(0, CH, rbody, 0)

    if has_sub:
        def body(lin_hbm, t1_hbm, i1_hbm, t2_hbm, i2_hbm, out_hbm,
                 i1_v, i2_v, a_v, b_v, c_v, sem0, sem1):
            wid = lax.axis_index("s") * NC + lax.axis_index("c")
            sems = (sem0, sem1)

            def issue(j, p):
                cid = wid + NW * j

                @pl.when(cid < n_chunks)
                def _():
                    off = cid * CH
                    pltpu.sync_copy(i1_hbm.at[pl.ds(off, CH)], i1_v.at[p])
                    pltpu.sync_copy(i2_hbm.at[pl.ds(off, CH)], i2_v.at[p])
                    pltpu.async_copy(t1_hbm.at[i1_v.at[p]], a_v.at[p], sems[p])
                    pltpu.async_copy(t2_hbm.at[i2_v.at[p]], b_v.at[p], sems[p])
                    pltpu.async_copy(lin_hbm.at[pl.ds(off, CH)], c_v.at[p],
                                     sems[p])

            def process(j, q):
                cid = wid + NW * j

                @pl.when(cid < n_chunks)
                def _():
                    off = cid * CH
                    pltpu.make_async_copy(t1_hbm.at[i1_v.at[q]], a_v.at[q],
                                          sems[q]).wait()
                    pltpu.make_async_copy(t2_hbm.at[i2_v.at[q]], b_v.at[q],
                                          sems[q]).wait()
                    pltpu.make_async_copy(lin_hbm.at[pl.ds(off, CH)],
                                          c_v.at[q], sems[q]).wait()
                    compute(c_v.at[q], a_v.at[q], b_v.at[q])
                    pltpu.sync_copy(c_v.at[q], out_hbm.at[pl.ds(off, CH)])

            def step(jj, carry):
                j = 2 * jj
                issue(j + 1, 1)
                process(j, 0)
                issue(j + 2, 0)
                process(j + 1, 1)
                return carry

            issue(0, 0)
            lax.fori_loop(0, (jmax + 1) // 2 + 1, step, 0)

        scratch = [
            pltpu.VMEM((2, CH), jnp.int32),
            pltpu.VMEM((2, CH), jnp.int32),
            pltpu.VMEM((2, CH, D), jnp.float32),
            pltpu.VMEM((2, CH, D), jnp.float32),
            pltpu.VMEM((2, CH, D), jnp.float32),
            pltpu.SemaphoreType.DMA,
            pltpu.SemaphoreType.DMA,
        ]
        args = (lin, t1, i1, t2, i2)
    else:
        def body(lin_hbm, t1_hbm, i1_hbm, out_hbm, i1_v, a_v, c_v, sem0, sem1):
            wid = lax.axis_index("s") * NC + lax.axis_index("c")
            sems = (sem0, sem1)

            def issue(j, p):
                cid = wid + NW * j

                @pl.when(cid < n_chunks)
                def _():
                    off = cid * CH
                    pltpu.sync_copy(i1_hbm.at[pl.ds(off, CH)], i1_v.at[p])
                    pltpu.async_copy(t1_hbm.at[i1_v.at[p]], a_v.at[p], sems[p])
                    pltpu.async_copy(lin_hbm.at[pl.ds(off, CH)], c_v.at[p],
                                     sems[p])

            def process(j, q):
                cid = wid + NW * j

                @pl.when(cid < n_chunks)
                def _():
                    off = cid * CH
                    pltpu.make_async_copy(t1_hbm.at[i1_v.at[q]], a_v.at[q],
                                          sems[q]).wait()
                    pltpu.make_async_copy(lin_hbm.at[pl.ds(off, CH)],
                                          c_v.at[q], sems[q]).wait()
                    compute(c_v.at[q], a_v.at[q], None)
                    pltpu.sync_copy(c_v.at[q], out_hbm.at[pl.ds(off, CH)])

            def step(jj, carry):
                j = 2 * jj
                issue(j + 1, 1)
                process(j, 0)
                issue(j + 2, 0)
                process(j + 1, 1)
                return carry

            issue(0, 0)
            lax.fori_loop(0, (jmax + 1) // 2 + 1, step, 0)

        scratch = [
            pltpu.VMEM((2, CH), jnp.int32),
            pltpu.VMEM((2, CH, D), jnp.float32),
            pltpu.VMEM((2, CH, D), jnp.float32),
            pltpu.SemaphoreType.DMA,
            pltpu.SemaphoreType.DMA,
        ]
        args = (lin, t1, i1)

    f = pl.kernel(
        body,
        out_type=jax.ShapeDtypeStruct((e_rows, D), jnp.float32),
        mesh=_sc_mesh(),
        scratch_types=scratch,
    )
    return f(*args)


def kernel(X, B, edge_index, rev_index, batch_vec, num_graphs, W_i, W_h, W_a):
    f32 = jnp.float32
    src = edge_index[0]
    rcv = edge_index[1]

    X_pad = jnp.zeros((NPAD, D), f32).at[:N_NODES].set(X)
    W_ixT = W_i[:, :D].T
    W_ibT = W_i[:, D:].T
    W_hT = W_h.T
    W_axT = W_a[:, :D].T
    W_ahT = W_a[:, D:].T
    z_node = jnp.zeros((NPAD // NS, D), f32)
    z_graph = jnp.zeros((NTG // NS, D), f32)

    XW = _mm(X_pad, W_ixT)
    BW = _mm(B, W_ibT)
    H0 = _fused_gather_call(BW, XW, src)
    H = H0
    for _ in range(STEPS):
        G = _mm(H, W_hT)
        parts = _scatter_add_call(G, rcv, z_node, NPAD).reshape(NC, NPAD, D)
        inc = _add2(parts, 512)
        H = _fused_gather_call(H0, inc, src, G, rev_index)

    parts_f = _scatter_add_call(H, rcv, z_node, NPAD).reshape(NC, NPAD, D)
    P = _mm2_relu(X_pad, W_axT, parts_f, W_ahT)
    b_pad = jnp.concatenate(
        [batch_vec, jnp.zeros((NPAD - N_NODES,), jnp.int32)]
    )
    parts_z = _scatter_add_call(P, b_pad, z_graph, NTG).reshape(NC, NTG, D)
    Z = _add2(parts_z, NTG)[:NUM_GRAPHS]
    return Z + jnp.asarray(num_graphs - NUM_GRAPHS, f32)


# speedup vs baseline: 3.1864x; 1.5027x over previous
import functools

import jax
import jax.numpy as jnp
from jax import lax
from jax.experimental import pallas as pl
from jax.experimental.pallas import tpu as pltpu
from jax.experimental.pallas import tpu_sc as plsc

N_NODES = 10000
N_EDGES = 320000
D = 128
D_EDGE = 16
STEPS = 3
NUM_GRAPHS = 64

NC = 2
NS = 16
NW = NC * NS
L = 16

CH = 128
NPAD = 10240
NTG = 128


def _sc_mesh():
    return plsc.VectorSubcoreMesh(
        core_axis_name="c", subcore_axis_name="s", num_cores=NC, num_subcores=NS
    )


def _mm(a, wt, block_m=512):
    M, K = a.shape
    N = wt.shape[1]

    def body(a_ref, w_ref, o_ref):
        o_ref[...] = jnp.dot(a_ref[...], w_ref[...],
                             preferred_element_type=jnp.float32)

    return pl.pallas_call(
        body,
        grid=(M // block_m,),
        in_specs=[
            pl.BlockSpec((block_m, K), lambda i: (i, 0)),
            pl.BlockSpec((K, N), lambda i: (0, 0)),
        ],
        out_specs=pl.BlockSpec((block_m, N), lambda i: (i, 0)),
        out_shape=jax.ShapeDtypeStruct((M, N), jnp.float32),
    )(a, wt)


def _readout(x, wx, parts, wh, bvec3, block_m=512):
    M, K = x.shape
    N = wx.shape[1]

    def body(x_ref, wx_ref, p_ref, wh_ref, b_ref, o_ref):
        i = pl.program_id(0)
        h = p_ref[0] + p_ref[1]
        acc = jnp.dot(x_ref[...], wx_ref[...],
                      preferred_element_type=jnp.float32)
        acc = acc + jnp.dot(h, wh_ref[...],
                            preferred_element_type=jnp.float32)
        pblk = jnp.maximum(acc, 0.0)
        b = b_ref[0, 0, :]
        oh = (b[:, None] == lax.broadcasted_iota(
            jnp.int32, (block_m, NUM_GRAPHS), 1)).astype(jnp.float32)
        z = lax.dot_general(oh, pblk, (((0,), (0,)), ((), ())),
                            preferred_element_type=jnp.float32)

        @pl.when(i == 0)
        def _():
            o_ref[...] = z

        @pl.when(i > 0)
        def _():
            o_ref[...] = o_ref[...] + z

    return pl.pallas_call(
        body,
        grid=(M // block_m,),
        in_specs=[
            pl.BlockSpec((block_m, K), lambda i: (i, 0)),
            pl.BlockSpec((K, N), lambda i: (0, 0)),
            pl.BlockSpec((2, block_m, D), lambda i: (0, i, 0)),
            pl.BlockSpec((D, N), lambda i: (0, 0)),
            pl.BlockSpec((1, 1, block_m), lambda i: (i, 0, 0)),
        ],
        out_specs=pl.BlockSpec((NUM_GRAPHS, N), lambda i: (0, 0)),
        out_shape=jax.ShapeDtypeStruct((NUM_GRAPHS, N), jnp.float32),
    )(x, wx, parts, wh, bvec3)


def _add2(parts, block_m):
    _, NT, D_ = parts.shape

    def body(p_ref, o_ref):
        o_ref[...] = p_ref[0] + p_ref[1]

    return pl.pallas_call(
        body,
        grid=(NT // block_m,),
        in_specs=[pl.BlockSpec((2, block_m, D_), lambda i: (0, i, 0))],
        out_specs=pl.BlockSpec((block_m, D_), lambda i: (i, 0)),
        out_shape=jax.ShapeDtypeStruct((NT, D_), jnp.float32),
    )(parts)


def _scatter_add_call(rows, idx, zeros, nt):
    e_rows = rows.shape[0]
    n_chunks = e_rows // CH
    jmax = -(-n_chunks // NW)
    rpt = nt // NS

    def body(rows_hbm, idx_hbm, zeros_hbm, out_hbm, idx_v, rows_v, table_sh,
             sem0, sem1):
        c = lax.axis_index("c")
        s = lax.axis_index("s")
        wid = s * NC + c
        pltpu.sync_copy(zeros_hbm, table_sh.at[pl.ds(s * rpt, rpt)])
        plsc.subcore_barrier()
        sems = (sem0, sem1)

        def issue(j, p):
            cid = wid + NW * j

            @pl.when(cid < n_chunks)
            def _():
                off = cid * CH
                pltpu.sync_copy(idx_hbm.at[pl.ds(off, CH)], idx_v.at[p])
                pltpu.async_copy(rows_hbm.at[pl.ds(off, CH)], rows_v.at[p],
                                 sems[p])

        def process(j, q):
            cid = wid + NW * j

            @pl.when(cid < n_chunks)
            def _():
                off = cid * CH
                pltpu.make_async_copy(rows_hbm.at[pl.ds(off, CH)],
                                      rows_v.at[q], sems[q]).wait()
                pltpu.sync_copy(rows_v.at[q], table_sh.at[idx_v.at[q]],
                                add=True)

        def step(jj, carry):
            j = 2 * jj
            issue(j + 1, 1)
            process(j, 0)
            issue(j + 2, 0)
            process(j + 1, 1)
            return carry

        issue(0, 0)
        lax.fori_loop(0, (jmax + 1) // 2 + 1, step, 0)
        plsc.subcore_barrier()
        pltpu.sync_copy(
            table_sh.at[pl.ds(s * rpt, rpt)],
            out_hbm.at[pl.ds(c * nt + s * rpt, rpt)],
        )

    f = pl.kernel(
        body,
        out_type=jax.ShapeDtypeStruct((NC * nt, D), jnp.float32),
        mesh=_sc_mesh(),
        scratch_types=[
            pltpu.VMEM((2, CH), jnp.int32),
            pltpu.VMEM((2, CH, D), jnp.float32),
            pltpu.VMEM_SHARED((nt, D), jnp.float32),
            pltpu.SemaphoreType.DMA,
            pltpu.SemaphoreType.DMA,
        ],
    )
    return f(rows, idx, zeros)


def _fused_gather_call(lin, t1, i1, t2=None, i2=None):
    e_rows = lin.shape[0]
    n_chunks = e_rows // CH
    jmax = -(-n_chunks // NW)
    has_sub = t2 is not None

    def compute(c_v, a_v, b_v):
        def rbody(r, carry):
            for q in range(D // L):
                sl = pl.ds(q * L, L)
                v = c_v[r, sl] + a_v[r, sl]
                if b_v is not None:
                    v = v - b_v[r, sl]
                c_v[r, sl] = jnp.maximum(v, 0.0)
            return carry

        lax.fori_loop(0, CH, rbody, 0)

    if has_sub:
        def body(lin_hbm, t1_hbm, i1_hbm, t2_hbm, i2_hbm, out_hbm,
                 i1_v, i2_v, a_v, b_v, c_v, sem0, sem1):
            wid = lax.axis_index("s") * NC + lax.axis_index("c")
            sems = (sem0, sem1)

            def issue(j, p):
                cid = wid + NW * j

                @pl.when(cid < n_chunks)
                def _():
                    off = cid * CH
                    pltpu.sync_copy(i1_hbm.at[pl.ds(off, CH)], i1_v.at[p])
                    pltpu.sync_copy(i2_hbm.at[pl.ds(off, CH)], i2_v.at[p])
                    pltpu.async_copy(t1_hbm.at[i1_v.at[p]], a_v.at[p], sems[p])
                    pltpu.async_copy(t2_hbm.at[i2_v.at[p]], b_v.at[p], sems[p])
                    pltpu.async_copy(lin_hbm.at[pl.ds(off, CH)], c_v.at[p],
                                     sems[p])

            def process(j, q):
                cid = wid + NW * j

                @pl.when(cid < n_chunks)
                def _():
                    off = cid * CH
                    pltpu.make_async_copy(t1_hbm.at[i1_v.at[q]], a_v.at[q],
                                          sems[q]).wait()
                    pltpu.make_async_copy(t2_hbm.at[i2_v.at[q]], b_v.at[q],
                                          sems[q]).wait()
                    pltpu.make_async_copy(lin_hbm.at[pl.ds(off, CH)],
                                          c_v.at[q], sems[q]).wait()
                    compute(c_v.at[q], a_v.at[q], b_v.at[q])
                    pltpu.sync_copy(c_v.at[q], out_hbm.at[pl.ds(off, CH)])

            def step(jj, carry):
                j = 2 * jj
                issue(j + 1, 1)
                process(j, 0)
                issue(j + 2, 0)
                process(j + 1, 1)
                return carry

            issue(0, 0)
            lax.fori_loop(0, (jmax + 1) // 2 + 1, step, 0)

        scratch = [
            pltpu.VMEM((2, CH), jnp.int32),
            pltpu.VMEM((2, CH), jnp.int32),
            pltpu.VMEM((2, CH, D), jnp.float32),
            pltpu.VMEM((2, CH, D), jnp.float32),
            pltpu.VMEM((2, CH, D), jnp.float32),
            pltpu.SemaphoreType.DMA,
            pltpu.SemaphoreType.DMA,
        ]
        args = (lin, t1, i1, t2, i2)
    else:
        def body(lin_hbm, t1_hbm, i1_hbm, out_hbm, i1_v, a_v, c_v, sem0, sem1):
            wid = lax.axis_index("s") * NC + lax.axis_index("c")
            sems = (sem0, sem1)

            def issue(j, p):
                cid = wid + NW * j

                @pl.when(cid < n_chunks)
                def _():
                    off = cid * CH
                    pltpu.sync_copy(i1_hbm.at[pl.ds(off, CH)], i1_v.at[p])
                    pltpu.async_copy(t1_hbm.at[i1_v.at[p]], a_v.at[p], sems[p])
                    pltpu.async_copy(lin_hbm.at[pl.ds(off, CH)], c_v.at[p],
                                     sems[p])

            def process(j, q):
                cid = wid + NW * j

                @pl.when(cid < n_chunks)
                def _():
                    off = cid * CH
                    pltpu.make_async_copy(t1_hbm.at[i1_v.at[q]], a_v.at[q],
                                          sems[q]).wait()
                    pltpu.make_async_copy(lin_hbm.at[pl.ds(off, CH)],
                                          c_v.at[q], sems[q]).wait()
                    compute(c_v.at[q], a_v.at[q], None)
                    pltpu.sync_copy(c_v.at[q], out_hbm.at[pl.ds(off, CH)])

            def step(jj, carry):
                j = 2 * jj
                issue(j + 1, 1)
                process(j, 0)
                issue(j + 2, 0)
                process(j + 1, 1)
                return carry

            issue(0, 0)
            lax.fori_loop(0, (jmax + 1) // 2 + 1, step, 0)

        scratch = [
            pltpu.VMEM((2, CH), jnp.int32),
            pltpu.VMEM((2, CH, D), jnp.float32),
            pltpu.VMEM((2, CH, D), jnp.float32),
            pltpu.SemaphoreType.DMA,
            pltpu.SemaphoreType.DMA,
        ]
        args = (lin, t1, i1)

    f = pl.kernel(
        body,
        out_type=jax.ShapeDtypeStruct((e_rows, D), jnp.float32),
        mesh=_sc_mesh(),
        scratch_types=scratch,
    )
    return f(*args)


def kernel(X, B, edge_index, rev_index, batch_vec, num_graphs, W_i, W_h, W_a):
    f32 = jnp.float32
    src = edge_index[0]
    rcv = edge_index[1]

    X_pad = jnp.zeros((NPAD, D), f32).at[:N_NODES].set(X)
    W_ixT = W_i[:, :D].T
    W_ibT = W_i[:, D:].T
    W_hT = W_h.T
    W_axT = W_a[:, :D].T
    W_ahT = W_a[:, D:].T
    z_node = jnp.zeros((NPAD // NS, D), f32)

    XW = _mm(X_pad, W_ixT, block_m=1024)
    BW = _mm(B, W_ibT, block_m=3200)
    H0 = _fused_gather_call(BW, XW, src)
    H = H0
    for _ in range(STEPS):
        G = _mm(H, W_hT, block_m=3200)
        parts = _scatter_add_call(G, rcv, z_node, NPAD).reshape(NC, NPAD, D)
        inc = _add2(parts, 512)
        H = _fused_gather_call(H0, inc, src, G, rev_index)

    parts_f = _scatter_add_call(H, rcv, z_node, NPAD).reshape(NC, NPAD, D)
    b_pad = jnp.concatenate(
        [batch_vec, jnp.zeros((NPAD - N_NODES,), jnp.int32)]
    )
    bvec3 = b_pad.reshape(NPAD // 512, 1, 512)
    Z = _readout(X_pad, W_axT, parts_f, W_ahT, bvec3, block_m=512)
    return Z + jnp.asarray(num_graphs - NUM_GRAPHS, f32)
